# baseline (speedup 1.0000x reference)
import functools

import jax
import jax.numpy as jnp
from jax import lax
from jax.experimental import pallas as pl
from jax.experimental.pallas import tpu as pltpu
from jax.experimental.pallas import tpu_sc as plsc

V = 1000
VP = 1024
N_ROWS = 1024 * 200
NC, NS, L = 2, 16, 16
NW = NC * NS
B_PER_W = N_ROWS // NW
CHUNK = 32
N_CHUNKS = B_PER_W // CHUNK
GROUPS = CHUNK // L


def _lz_body(table_ref, lz_ref):
    x = table_ref[...]
    m = jnp.max(x, axis=1, keepdims=True)
    s = jnp.sum(jnp.exp(x - m), axis=1, keepdims=True)
    lz_ref[...] = m + jnp.log(s)


_lz_call = pl.pallas_call(
    _lz_body,
    out_shape=jax.ShapeDtypeStruct((V, 1), jnp.float32),
)


BIAS = 7.5


def _loss_body(part_ref, out_ref):
    s = jnp.sum(part_ref[...], dtype=jnp.float32)
    out_ref[...] = (s * (1.0 / N_ROWS) + BIAS).reshape(1, 1)


_loss_call = pl.pallas_call(
    _loss_body,
    out_shape=jax.ShapeDtypeStruct((1, 1), jnp.float32),
)


_mesh = plsc.VectorSubcoreMesh(core_axis_name="c", subcore_axis_name="s")


@functools.partial(
    pl.kernel,
    mesh=_mesh,
    out_type=[
        jax.ShapeDtypeStruct((N_ROWS, V), jnp.float32),
        jax.ShapeDtypeStruct((N_ROWS, 128), jnp.float32),
        jax.ShapeDtypeStruct((NW, L), jnp.float32),
    ],
    scratch_types=[
        pltpu.VMEM((CHUNK,), jnp.int32),
        pltpu.VMEM((CHUNK,), jnp.int32),
        pltpu.VMEM((CHUNK,), jnp.int32),
        pltpu.VMEM((CHUNK,), jnp.int32),
        pltpu.VMEM((CHUNK,), jnp.int32),
        pltpu.VMEM((CHUNK,), jnp.int32),
        pltpu.VMEM((CHUNK, VP), jnp.float32),
        pltpu.VMEM((CHUNK, VP), jnp.float32),
        pltpu.VMEM((CHUNK,), jnp.float32),
        pltpu.VMEM((CHUNK,), jnp.float32),
        pltpu.VMEM((CHUNK,), jnp.float32),
        pltpu.VMEM((CHUNK,), jnp.float32),
        pltpu.VMEM((L,), jnp.float32),
        pltpu.SemaphoreType.DMA,
        pltpu.SemaphoreType.DMA,
        pltpu.SemaphoreType.DMA,
        pltpu.SemaphoreType.DMA,
        pltpu.SemaphoreType.DMA,
        pltpu.SemaphoreType.DMA,
    ],
)
def _sc_gather(tablep_hbm, flat_hbm, inp_hbm, tgt_hbm, lz_hbm,
               out_hbm, tail_hbm, part_hbm,
               idx0, idx1, tgt0, tgt1, comb0, comb1, rows0, rows1,
               picked0, picked1, lzg0, lzg1, part_v,
               gsem0, gsem1, osem0, osem1, psem0, psem1):
    idxs, tgts, combs = [idx0, idx1], [tgt0, tgt1], [comb0, comb1]
    rows, pickeds, lzgs = [rows0, rows1], [picked0, picked1], [lzg0, lzg1]
    gsems, osems, psems = [gsem0, gsem1], [osem0, osem1], [psem0, psem1]
    wid = lax.axis_index("s") * NC + lax.axis_index("c")
    base = wid * B_PER_W

    for b in range(2):
        start = base + b * CHUNK
        pltpu.sync_copy(inp_hbm.at[pl.ds(start, CHUNK)], idxs[b])
        pltpu.sync_copy(tgt_hbm.at[pl.ds(start, CHUNK)], tgts[b])
        pltpu.async_copy(tablep_hbm.at[idxs[b]], rows[b], gsems[b])

    def loop_body(i, acc):
        for b in range(2):
            c = 2 * i + b
            start = base + c * CHUNK
            for g in range(GROUPS):
                i16 = idxs[b][pl.ds(g * L, L)]
                t16 = tgts[b][pl.ds(g * L, L)]
                combs[b][pl.ds(g * L, L)] = i16 * V + t16
            pltpu.async_copy(flat_hbm.at[combs[b]], pickeds[b], psems[b])
            pltpu.async_copy(lz_hbm.at[idxs[b]], lzgs[b], psems[b])
            pltpu.make_async_copy(
                tablep_hbm.at[idxs[b]], rows[b], gsems[b]).wait()
            pltpu.async_copy(
                rows[b].at[:, pl.ds(0, 896)],
                out_hbm.at[pl.ds(start, CHUNK), pl.ds(0, 896)], osems[b])
            pltpu.async_copy(
                rows[b].at[:, pl.ds(896, 128)],
                tail_hbm.at[pl.ds(start, CHUNK)], osems[b])
            pltpu.make_async_copy(
                flat_hbm.at[combs[b]], pickeds[b], psems[b]).wait()
            pltpu.make_async_copy(
                lz_hbm.at[idxs[b]], lzgs[b], psems[b]).wait()
            for g in range(GROUPS):
                acc = acc + (lzgs[b][pl.ds(g * L, L)]
                             - pickeds[b][pl.ds(g * L, L)] - BIAS)
            pltpu.make_async_copy(
                rows[b].at[:, pl.ds(0, 896)],
                out_hbm.at[pl.ds(start, CHUNK), pl.ds(0, 896)], osems[b]).wait()
            pltpu.make_async_copy(
                rows[b].at[:, pl.ds(896, 128)],
                tail_hbm.at[pl.ds(start, CHUNK)], osems[b]).wait()

            @pl.when(c + 2 < N_CHUNKS)
            def _():
                nstart = base + (c + 2) * CHUNK
                pltpu.sync_copy(inp_hbm.at[pl.ds(nstart, CHUNK)], idxs[b])
                pltpu.sync_copy(tgt_hbm.at[pl.ds(nstart, CHUNK)], tgts[b])
                pltpu.async_copy(tablep_hbm.at[idxs[b]], rows[b], gsems[b])
        return acc

    acc = lax.fori_loop(0, N_CHUNKS // 2, loop_body,
                        jnp.zeros((L,), jnp.float32))
    part_v[...] = acc
    pltpu.sync_copy(part_v, part_hbm.at[wid])


MBLK = 2048
N_MBLK = N_ROWS // MBLK


def _merge_body(tail_ref, main_ref, out_ref):
    del main_ref
    out_ref[...] = tail_ref[...]


_merge_call = pl.pallas_call(
    _merge_body,
    grid=(N_MBLK,),
    in_specs=[
        pl.BlockSpec((MBLK, 128), lambda i: (i, 0)),
        pl.BlockSpec(memory_space=pltpu.MemorySpace.HBM),
    ],
    out_specs=pl.BlockSpec((MBLK, 128), lambda i: (i, 7)),
    out_shape=jax.ShapeDtypeStruct((N_ROWS, V), jnp.float32),
    input_output_aliases={1: 0},
)


def kernel(table, inputs, targets):
    inp = inputs.reshape(-1).astype(jnp.int32)
    tgt = targets.reshape(-1).astype(jnp.int32)
    tablep = jnp.pad(table, ((0, 0), (0, VP - V)))
    flat = table.reshape(-1)
    lz = _lz_call(table)[:, 0]
    main, tail, part = _sc_gather(tablep, flat, inp, tgt, lz)
    loss = _loss_call(part)[0, 0]
    logits2 = _merge_call(tail, main)
    return (logits2, loss)


# speedup vs baseline: 1.0025x; 1.0025x over previous
import functools

import jax
import jax.numpy as jnp
from jax import lax
from jax.experimental import pallas as pl
from jax.experimental.pallas import tpu as pltpu
from jax.experimental.pallas import tpu_sc as plsc

V = 1000
VP = 1024
N_ROWS = 1024 * 200
NC, NS, L = 2, 16, 16
NW = NC * NS
B_PER_W = N_ROWS // NW
CHUNK = 32
N_CHUNKS = B_PER_W // CHUNK
GROUPS = CHUNK // L


def _lz_body(table_ref, lz_ref):
    x = table_ref[...]
    m = jnp.max(x, axis=1, keepdims=True)
    s = jnp.sum(jnp.exp(x - m), axis=1, keepdims=True)
    lz_ref[...] = m + jnp.log(s)


_lz_call = pl.pallas_call(
    _lz_body,
    out_shape=jax.ShapeDtypeStruct((V, 1), jnp.float32),
)


BIAS = 7.5


def _loss_body(part_ref, out_ref):
    s = jnp.sum(part_ref[...], dtype=jnp.float32)
    out_ref[...] = (s * (1.0 / N_ROWS) + BIAS).reshape(1, 1)


_loss_call = pl.pallas_call(
    _loss_body,
    out_shape=jax.ShapeDtypeStruct((1, 1), jnp.float32),
)


_mesh = plsc.VectorSubcoreMesh(core_axis_name="c", subcore_axis_name="s")


@functools.partial(
    pl.kernel,
    mesh=_mesh,
    out_type=[
        jax.ShapeDtypeStruct((N_ROWS, V), jnp.float32),
        jax.ShapeDtypeStruct((N_ROWS, 128), jnp.float32),
        jax.ShapeDtypeStruct((NW, L), jnp.float32),
    ],
    scratch_types=[
        pltpu.VMEM((CHUNK,), jnp.int32),
        pltpu.VMEM((CHUNK,), jnp.int32),
        pltpu.VMEM((CHUNK,), jnp.int32),
        pltpu.VMEM((CHUNK,), jnp.int32),
        pltpu.VMEM((CHUNK,), jnp.int32),
        pltpu.VMEM((CHUNK,), jnp.int32),
        pltpu.VMEM((CHUNK, VP), jnp.float32),
        pltpu.VMEM((CHUNK, VP), jnp.float32),
        pltpu.VMEM((CHUNK,), jnp.float32),
        pltpu.VMEM((CHUNK,), jnp.float32),
        pltpu.VMEM((CHUNK,), jnp.float32),
        pltpu.VMEM((CHUNK,), jnp.float32),
        pltpu.VMEM((L,), jnp.float32),
        pltpu.SemaphoreType.DMA,
        pltpu.SemaphoreType.DMA,
        pltpu.SemaphoreType.DMA,
        pltpu.SemaphoreType.DMA,
        pltpu.SemaphoreType.DMA,
        pltpu.SemaphoreType.DMA,
    ],
)
def _sc_gather(tablep_hbm, flat_hbm, inp_hbm, tgt_hbm, lz_hbm,
               out_hbm, tail_hbm, part_hbm,
               idx0, idx1, tgt0, tgt1, comb0, comb1, rows0, rows1,
               picked0, picked1, lzg0, lzg1, part_v,
               gsem0, gsem1, osem0, osem1, psem0, psem1):
    idxs, tgts, combs = [idx0, idx1], [tgt0, tgt1], [comb0, comb1]
    rows, pickeds, lzgs = [rows0, rows1], [picked0, picked1], [lzg0, lzg1]
    gsems, osems, psems = [gsem0, gsem1], [osem0, osem1], [psem0, psem1]
    wid = lax.axis_index("s") * NC + lax.axis_index("c")
    base = wid * B_PER_W

    def stage(b, start):
        pltpu.sync_copy(inp_hbm.at[pl.ds(start, CHUNK)], idxs[b])
        pltpu.sync_copy(tgt_hbm.at[pl.ds(start, CHUNK)], tgts[b])
        for g in range(GROUPS):
            i16 = idxs[b][pl.ds(g * L, L)]
            t16 = tgts[b][pl.ds(g * L, L)]
            combs[b][pl.ds(g * L, L)] = i16 * V + t16
        pltpu.async_copy(flat_hbm.at[combs[b]], pickeds[b], psems[b])
        pltpu.async_copy(lz_hbm.at[idxs[b]], lzgs[b], psems[b])
        pltpu.async_copy(tablep_hbm.at[idxs[b]], rows[b], gsems[b])

    for b in range(2):
        stage(b, base + b * CHUNK)

    def loop_body(i, acc):
        for b in range(2):
            c = 2 * i + b
            start = base + c * CHUNK
            pltpu.make_async_copy(
                tablep_hbm.at[idxs[b]], rows[b], gsems[b]).wait()
            pltpu.async_copy(
                rows[b].at[:, pl.ds(0, 896)],
                out_hbm.at[pl.ds(start, CHUNK), pl.ds(0, 896)], osems[b])
            pltpu.async_copy(
                rows[b].at[:, pl.ds(896, 128)],
                tail_hbm.at[pl.ds(start, CHUNK)], osems[b])
            pltpu.make_async_copy(
                flat_hbm.at[combs[b]], pickeds[b], psems[b]).wait()
            pltpu.make_async_copy(
                lz_hbm.at[idxs[b]], lzgs[b], psems[b]).wait()
            for g in range(GROUPS):
                acc = acc + (lzgs[b][pl.ds(g * L, L)]
                             - pickeds[b][pl.ds(g * L, L)] - BIAS)
            pltpu.make_async_copy(
                rows[b].at[:, pl.ds(0, 896)],
                out_hbm.at[pl.ds(start, CHUNK), pl.ds(0, 896)], osems[b]).wait()
            pltpu.make_async_copy(
                rows[b].at[:, pl.ds(896, 128)],
                tail_hbm.at[pl.ds(start, CHUNK)], osems[b]).wait()

            @pl.when(c + 2 < N_CHUNKS)
            def _():
                stage(b, base + (c + 2) * CHUNK)
        return acc

    acc = lax.fori_loop(0, N_CHUNKS // 2, loop_body,
                        jnp.zeros((L,), jnp.float32))
    part_v[...] = acc
    pltpu.sync_copy(part_v, part_hbm.at[wid])


MBLK = 2048
N_MBLK = N_ROWS // MBLK


def _merge_body(tail_ref, main_ref, out_ref):
    del main_ref
    out_ref[...] = tail_ref[...]


_merge_call = pl.pallas_call(
    _merge_body,
    grid=(N_MBLK,),
    in_specs=[
        pl.BlockSpec((MBLK, 128), lambda i: (i, 0)),
        pl.BlockSpec(memory_space=pltpu.MemorySpace.HBM),
    ],
    out_specs=pl.BlockSpec((MBLK, 128), lambda i: (i, 7)),
    out_shape=jax.ShapeDtypeStruct((N_ROWS, V), jnp.float32),
    input_output_aliases={1: 0},
)


def kernel(table, inputs, targets):
    inp = inputs.reshape(-1).astype(jnp.int32)
    tgt = targets.reshape(-1).astype(jnp.int32)
    tablep = jnp.pad(table, ((0, 0), (0, VP - V)))
    flat = table.reshape(-1)
    lz = _lz_call(table)[:, 0]
    main, tail, part = _sc_gather(tablep, flat, inp, tgt, lz)
    loss = _loss_call(part)[0, 0]
    logits2 = _merge_call(tail, main)
    return (logits2, loss)


# speedup vs baseline: 1.2383x; 1.2352x over previous
import functools

import jax
import jax.numpy as jnp
from jax import lax
from jax.experimental import pallas as pl
from jax.experimental.pallas import tpu as pltpu
from jax.experimental.pallas import tpu_sc as plsc

V = 1000
VP = 1024
N_ROWS = 1024 * 200
NC, NS, L = 2, 16, 16
NW = NC * NS
B_PER_W = N_ROWS // NW
CHUNK = 40
N_CHUNKS = B_PER_W // CHUNK
HBLK = 1024
N_HBLK = N_ROWS // HBLK


def _lz_body(table_ref, lz_ref):
    x = table_ref[...]
    m = jnp.max(x, axis=1, keepdims=True)
    s = jnp.sum(jnp.exp(x - m), axis=1, keepdims=True)
    lz_ref[...] = m + jnp.log(s)


_lz_call = pl.pallas_call(
    _lz_body,
    out_shape=jax.ShapeDtypeStruct((V, 1), jnp.float32),
)


def _hist_body(inp_ref, tgt_ref, h_ref):
    i = pl.program_id(0)

    @pl.when(i == 0)
    def _():
        h_ref[...] = jnp.zeros_like(h_ref)

    cols = lax.broadcasted_iota(jnp.int32, (HBLK, VP), 1)
    iv = inp_ref[0, 0, :]
    tv = tgt_ref[0, 0, :]
    a = (iv[:, None] == cols).astype(jnp.int8)
    b = (tv[:, None] == cols).astype(jnp.int8)
    h_ref[...] += lax.dot_general(
        a, b, (((0,), (0,)), ((), ())),
        preferred_element_type=jnp.int32)


_hist_call = pl.pallas_call(
    _hist_body,
    grid=(N_HBLK,),
    in_specs=[
        pl.BlockSpec((1, 1, HBLK), lambda i: (i, 0, 0)),
        pl.BlockSpec((1, 1, HBLK), lambda i: (i, 0, 0)),
    ],
    out_specs=pl.BlockSpec((VP, VP), lambda i: (0, 0)),
    out_shape=jax.ShapeDtypeStruct((VP, VP), jnp.int32),
)


def _loss_body(h_ref, table_ref, lz_ref, out_ref):
    h = h_ref[...].astype(jnp.float32)
    row_cnt = jnp.sum(h, axis=1)[:V]
    lz_sum = jnp.sum(row_cnt * lz_ref[:, 0])
    picked_sum = jnp.sum(h[:V, :V] * table_ref[...])
    out_ref[...] = ((lz_sum - picked_sum) * (1.0 / N_ROWS)).reshape(1, 1)


_loss_call = pl.pallas_call(
    _loss_body,
    out_shape=jax.ShapeDtypeStruct((1, 1), jnp.float32),
)


_mesh = plsc.VectorSubcoreMesh(core_axis_name="c", subcore_axis_name="s")


@functools.partial(
    pl.kernel,
    mesh=_mesh,
    out_type=[
        jax.ShapeDtypeStruct((N_ROWS, V), jnp.float32),
        jax.ShapeDtypeStruct((N_ROWS, 128), jnp.float32),
    ],
    scratch_types=[
        pltpu.VMEM((B_PER_W,), jnp.int32),
        pltpu.VMEM((CHUNK, VP), jnp.float32),
        pltpu.VMEM((CHUNK, VP), jnp.float32),
        pltpu.SemaphoreType.DMA,
        pltpu.SemaphoreType.DMA,
        pltpu.SemaphoreType.DMA,
        pltpu.SemaphoreType.DMA,
    ],
)
def _sc_gather(tablep_hbm, inp_hbm, out_hbm, tail_hbm,
               idx_all, rows0, rows1, gsem0, gsem1, osem0, osem1):
    rows = [rows0, rows1]
    gsems, osems = [gsem0, gsem1], [osem0, osem1]
    wid = lax.axis_index("s") * NC + lax.axis_index("c")
    base = wid * B_PER_W

    pltpu.sync_copy(inp_hbm.at[pl.ds(base, B_PER_W)], idx_all)

    def idxv(c):
        return idx_all.at[pl.ds(c * CHUNK, CHUNK)]

    for b in range(2):
        pltpu.async_copy(tablep_hbm.at[idxv(b)], rows[b], gsems[b])

    def loop_body(i, carry):
        for b in range(2):
            c = 2 * i + b
            start = base + c * CHUNK
            pltpu.make_async_copy(
                tablep_hbm.at[idxv(c)], rows[b], gsems[b]).wait()
            pltpu.async_copy(
                rows[b].at[:, pl.ds(0, 896)],
                out_hbm.at[pl.ds(start, CHUNK), pl.ds(0, 896)], osems[b])
            pltpu.async_copy(
                rows[b].at[:, pl.ds(896, 128)],
                tail_hbm.at[pl.ds(start, CHUNK)], osems[b])
            pltpu.make_async_copy(
                rows[b].at[:, pl.ds(0, 896)],
                out_hbm.at[pl.ds(start, CHUNK), pl.ds(0, 896)], osems[b]).wait()
            pltpu.make_async_copy(
                rows[b].at[:, pl.ds(896, 128)],
                tail_hbm.at[pl.ds(start, CHUNK)], osems[b]).wait()

            @pl.when(c + 2 < N_CHUNKS)
            def _():
                pltpu.async_copy(
                    tablep_hbm.at[idxv(c + 2)], rows[b], gsems[b])
        return carry

    lax.fori_loop(0, N_CHUNKS // 2, loop_body, 0)


def kernel(table, inputs, targets):
    inp = inputs.reshape(-1).astype(jnp.int32)
    tgt = targets.reshape(-1).astype(jnp.int32)
    tablep = jnp.pad(table, ((0, 0), (0, VP - V)))
    lz = _lz_call(table)
    h = _hist_call(inp.reshape(N_HBLK, 1, HBLK), tgt.reshape(N_HBLK, 1, HBLK))
    main, tail = _sc_gather(tablep, inp)
    loss = _loss_call(h, table, lz)[0, 0]
    logits2 = lax.dynamic_update_slice(main, tail[:, :V - 896], (0, 896))
    return (logits2, loss)
